# baseline (speedup 1.0000x reference)
import functools

import jax
import jax.numpy as jnp
from jax import lax
from jax.experimental import pallas as pl
from jax.experimental.pallas import tpu as pltpu
from jax.experimental.pallas import tpu_sc as plsc

_LANES = 128
_H_PAD = 1024
_O_PAD = 128
_BB = 8192


def _gather_body(rows_half, emb_hbm, idx1_hbm, idx2_hbm, out_hbm,
                 idx_v, rows0, rows1, sem0, sem1):
    nc = 2
    n_half = rows_half * 32
    wid = lax.axis_index("s") * nc + lax.axis_index("c")
    base = wid * rows_half
    pltpu.sync_copy(idx1_hbm.at[pl.ds(base, rows_half)],
                    idx_v.at[pl.ds(0, rows_half)])
    pltpu.sync_copy(idx2_hbm.at[pl.ds(base, rows_half)],
                    idx_v.at[pl.ds(rows_half, rows_half)])
    total = 2 * rows_half
    bufs = (rows0, rows1)
    sems = (sem0, sem1)
    copies = [None] * total
    copies[0] = pltpu.async_copy(emb_hbm.at[idx_v.at[0]], bufs[0], sems[0])
    for j in range(total):
        if j + 1 < total:
            copies[j + 1] = pltpu.async_copy(
                emb_hbm.at[idx_v.at[j + 1]], bufs[(j + 1) % 2], sems[(j + 1) % 2])
        out_row = base + j if j < rows_half else n_half + base + (j - rows_half)
        copies[j].wait()
        pltpu.sync_copy(bufs[j % 2], out_hbm.at[pl.ds(out_row * _LANES, _LANES)])


def _sc_gather(emb, idx1, idx2):
    b, d = idx1.shape[0], emb.shape[1]
    info = plsc.get_sparse_core_info()
    nw = info.num_cores * info.num_subcores
    rows_half = b // (nw * _LANES)
    i1 = idx1.astype(jnp.int32).reshape(nw * rows_half, _LANES)
    i2 = idx2.astype(jnp.int32).reshape(nw * rows_half, _LANES)
    mesh = plsc.VectorSubcoreMesh(core_axis_name="c", subcore_axis_name="s")
    f = pl.kernel(
        functools.partial(_gather_body, rows_half),
        out_type=jax.ShapeDtypeStruct((2 * b, d), jnp.float32),
        mesh=mesh,
        scratch_types=[
            pltpu.VMEM((2 * rows_half, _LANES), jnp.int32),
            pltpu.VMEM((_LANES, d), jnp.float32),
            pltpu.VMEM((_LANES, d), jnp.float32),
            pltpu.SemaphoreType.DMA,
            pltpu.SemaphoreType.DMA,
        ],
    )
    return f(emb, i1, i2)


def _mlp_body(o_dim, g_ref, w1a_ref, w1b_ref, b1_ref, w2_ref, b2_ref, out_ref):
    g1 = g_ref[0].astype(jnp.bfloat16)
    g2 = g_ref[1].astype(jnp.bfloat16)
    h = jnp.dot(g1, w1a_ref[...], preferred_element_type=jnp.float32)
    h = h + jnp.dot(g2, w1b_ref[...], preferred_element_type=jnp.float32)
    h = jnp.maximum(h + b1_ref[...], 0.0).astype(jnp.bfloat16)
    o = jnp.dot(h, w2_ref[...], preferred_element_type=jnp.float32) + b2_ref[...]
    m = jnp.max(o, axis=1, keepdims=True)
    e = jnp.exp(o - m)
    out_ref[...] = e * (1.0 / jnp.sum(e, axis=1, keepdims=True))


def _mlp(g3, w1a, w1b, b1p, w2p, b2p, o_dim, interpret=False):
    _, b, d = g3.shape
    return pl.pallas_call(
        functools.partial(_mlp_body, o_dim),
        grid=(b // _BB,),
        in_specs=[
            pl.BlockSpec((2, _BB, d), lambda i: (0, i, 0)),
            pl.BlockSpec((d, _H_PAD), lambda i: (0, 0)),
            pl.BlockSpec((d, _H_PAD), lambda i: (0, 0)),
            pl.BlockSpec((1, _H_PAD), lambda i: (0, 0)),
            pl.BlockSpec((_H_PAD, _O_PAD), lambda i: (0, 0)),
            pl.BlockSpec((1, _O_PAD), lambda i: (0, 0)),
        ],
        out_specs=pl.BlockSpec((_BB, _O_PAD), lambda i: (i, 0)),
        out_shape=jax.ShapeDtypeStruct((b, _O_PAD), jnp.float32),
        interpret=interpret,
    )(g3, w1a, w1b, b1p, w2p, b2p)[:, :o_dim]


def kernel(in1, in2, emb, W1, b1, W2, b2):
    b = in1.shape[0]
    d = emb.shape[1]
    h = W1.shape[1]
    o = W2.shape[1]
    g = _sc_gather(emb, in1, in2)
    g3 = g.reshape(2, b, d)
    w1a = jnp.pad(W1[:d], ((0, 0), (0, _H_PAD - h))).astype(jnp.bfloat16)
    w1b = jnp.pad(W1[d:], ((0, 0), (0, _H_PAD - h))).astype(jnp.bfloat16)
    b1p = jnp.pad(b1, (0, _H_PAD - h)).reshape(1, _H_PAD).astype(jnp.bfloat16)
    w2p = jnp.pad(W2, ((0, _H_PAD - h), (0, _O_PAD - o))).astype(jnp.bfloat16)
    b2p = jnp.pad(b2, (0, _O_PAD - o), constant_values=-1e30).reshape(1, _O_PAD)
    return _mlp(g3, w1a, w1b, b1p, w2p, b2p, o)


# speedup vs baseline: 1.0328x; 1.0328x over previous
import functools

import jax
import jax.numpy as jnp
from jax import lax
from jax.experimental import pallas as pl
from jax.experimental.pallas import tpu as pltpu
from jax.experimental.pallas import tpu_sc as plsc

_LANES = 128
_H_PAD = 1024
_O_PAD = 128
_BB = 4096


_NBUF = 7


def _gather_body(rows_half, emb_hbm, idx1_hbm, idx2_hbm, out_hbm,
                 idx_v, *scratch):
    bufs = scratch[:_NBUF]
    gsems = scratch[_NBUF:2 * _NBUF]
    wsems = scratch[2 * _NBUF:3 * _NBUF]
    nc = 2
    n_half = rows_half * 32
    wid = lax.axis_index("s") * nc + lax.axis_index("c")
    base = wid * rows_half
    pltpu.sync_copy(idx1_hbm.at[pl.ds(base, rows_half)],
                    idx_v.at[pl.ds(0, rows_half)])
    pltpu.sync_copy(idx2_hbm.at[pl.ds(base, rows_half)],
                    idx_v.at[pl.ds(rows_half, rows_half)])
    total = 2 * rows_half
    gathers = [None] * total
    writes = [None] * total
    for j in range(min(_NBUF, total)):
        gathers[j] = pltpu.async_copy(emb_hbm.at[idx_v.at[j]], bufs[j], gsems[j])
    for j in range(total):
        b = j % _NBUF
        out_row = base + j if j < rows_half else n_half + base + (j - rows_half)
        gathers[j].wait()
        writes[j] = pltpu.async_copy(
            bufs[b], out_hbm.at[pl.ds(out_row * _LANES, _LANES)], wsems[b])
        nxt = j + _NBUF
        if nxt < total:
            writes[j].wait()
            gathers[nxt] = pltpu.async_copy(
                emb_hbm.at[idx_v.at[nxt]], bufs[b], gsems[b])
            writes[j] = None
    for j in range(total):
        if writes[j] is not None:
            writes[j].wait()


def _sc_gather(emb, idx1, idx2):
    b, d = idx1.shape[0], emb.shape[1]
    info = plsc.get_sparse_core_info()
    nw = info.num_cores * info.num_subcores
    rows_half = b // (nw * _LANES)
    i1 = idx1.astype(jnp.int32).reshape(nw * rows_half, _LANES)
    i2 = idx2.astype(jnp.int32).reshape(nw * rows_half, _LANES)
    mesh = plsc.VectorSubcoreMesh(core_axis_name="c", subcore_axis_name="s")
    f = pl.kernel(
        functools.partial(_gather_body, rows_half),
        out_type=jax.ShapeDtypeStruct((2 * b, d), jnp.float32),
        mesh=mesh,
        scratch_types=(
            [pltpu.VMEM((2 * rows_half, _LANES), jnp.int32)]
            + [pltpu.VMEM((_LANES, d), jnp.float32)] * _NBUF
            + [pltpu.SemaphoreType.DMA] * (2 * _NBUF)
        ),
    )
    return f(emb, i1, i2)


def _mlp_body(o_dim, g_ref, w1a_ref, w1b_ref, b1_ref, w2_ref, b2_ref, out_ref):
    g1 = g_ref[0].astype(jnp.bfloat16)
    g2 = g_ref[1].astype(jnp.bfloat16)
    h = jnp.dot(g1, w1a_ref[...], preferred_element_type=jnp.float32)
    h = h + jnp.dot(g2, w1b_ref[...], preferred_element_type=jnp.float32)
    h = jnp.maximum(h + b1_ref[...], 0.0).astype(jnp.bfloat16)
    o = jnp.dot(h, w2_ref[...], preferred_element_type=jnp.float32) + b2_ref[...]
    m = jnp.max(o, axis=1, keepdims=True)
    e = jnp.exp(o - m)
    out_ref[...] = e * (1.0 / jnp.sum(e, axis=1, keepdims=True))


def _mlp(g3, w1a, w1b, b1p, w2p, b2p, o_dim, interpret=False):
    _, b, d = g3.shape
    return pl.pallas_call(
        functools.partial(_mlp_body, o_dim),
        grid=(b // _BB,),
        in_specs=[
            pl.BlockSpec((2, _BB, d), lambda i: (0, i, 0)),
            pl.BlockSpec((d, _H_PAD), lambda i: (0, 0)),
            pl.BlockSpec((d, _H_PAD), lambda i: (0, 0)),
            pl.BlockSpec((1, _H_PAD), lambda i: (0, 0)),
            pl.BlockSpec((_H_PAD, _O_PAD), lambda i: (0, 0)),
            pl.BlockSpec((1, _O_PAD), lambda i: (0, 0)),
        ],
        out_specs=pl.BlockSpec((_BB, _O_PAD), lambda i: (i, 0)),
        out_shape=jax.ShapeDtypeStruct((b, _O_PAD), jnp.float32),
        interpret=interpret,
    )(g3, w1a, w1b, b1p, w2p, b2p)[:, :o_dim]


def kernel(in1, in2, emb, W1, b1, W2, b2):
    b = in1.shape[0]
    d = emb.shape[1]
    h = W1.shape[1]
    o = W2.shape[1]
    g = _sc_gather(emb, in1, in2)
    g3 = g.reshape(2, b, d)
    w1a = jnp.pad(W1[:d], ((0, 0), (0, _H_PAD - h))).astype(jnp.bfloat16)
    w1b = jnp.pad(W1[d:], ((0, 0), (0, _H_PAD - h))).astype(jnp.bfloat16)
    b1p = jnp.pad(b1, (0, _H_PAD - h)).reshape(1, _H_PAD).astype(jnp.bfloat16)
    w2p = jnp.pad(W2, ((0, _H_PAD - h), (0, _O_PAD - o))).astype(jnp.bfloat16)
    b2p = jnp.pad(b2, (0, _O_PAD - o), constant_values=-1e30).reshape(1, _O_PAD)
    return _mlp(g3, w1a, w1b, b1p, w2p, b2p, o)
